# Initial kernel scaffold; baseline (speedup 1.0000x reference)
import jax
import jax.numpy as jnp
from jax.experimental import pallas as pl


def kernel(x, edge_index, W1, b1, W2, b2):
    raise NotImplementedError("write your pallas kernel here")


# speedup vs baseline: 31.7985x; 31.7985x over previous
import functools

import jax
import jax.numpy as jnp
from jax import lax
from jax.experimental import pallas as pl
from jax.experimental.pallas import tpu as pltpu
from jax.experimental.pallas import tpu_sc as plsc

N = 10000
E = 320000
D = 128
H = 128

NC = 2
NS = 16
NW = NC * NS
L = 16

NPAD = 10240
RPT = NPAD // NS
K = 80
ET = E // NW
NCH = ET // K
BR = 1024


def _sc_mesh():
    return plsc.VectorSubcoreMesh(core_axis_name="c", subcore_axis_name="s",
                                  num_cores=NC, num_subcores=NS)


@functools.partial(
    pl.kernel,
    mesh=_sc_mesh(),
    out_type=jax.ShapeDtypeStruct((NC, NPAD, L), jnp.float32),
    scratch_types=[
        pltpu.VMEM_SHARED((NPAD, L), jnp.float32),
        pltpu.VMEM((NCH, K), jnp.int32),
        pltpu.VMEM((K, L), jnp.float32),
        pltpu.VMEM((RPT, L), jnp.float32),
        pltpu.SemaphoreType.DMA,
    ],
    compiler_params=pltpu.CompilerParams(use_tc_tiling_on_sc=False),
)
def _deg_kernel(dst_hbm, out_hbm, acc_sh, idx_v, ones_v, z_v, sem):
    c = lax.axis_index("c")
    s = lax.axis_index("s")
    tile = c * NS + s

    def fill_z(i, _):
        z_v[i] = jnp.zeros((L,), jnp.float32)
        return 0

    lax.fori_loop(0, RPT, fill_z, 0)

    def fill_ones(i, _):
        ones_v[i] = jnp.full((L,), 1.0, jnp.float32)
        return 0

    lax.fori_loop(0, K, fill_ones, 0)

    pltpu.sync_copy(z_v, acc_sh.at[pl.ds(s * RPT, RPT)])
    plsc.subcore_barrier()

    pltpu.sync_copy(dst_hbm.at[tile], idx_v)

    def body(j, _):
        pltpu.sync_copy(ones_v, acc_sh.at[idx_v.at[j]], add=True)
        return 0

    lax.fori_loop(0, NCH, body, 0)
    plsc.subcore_barrier()

    pltpu.sync_copy(acc_sh.at[pl.ds(s * RPT, RPT)],
                    out_hbm.at[c, pl.ds(s * RPT, RPT)])


@functools.partial(
    pl.kernel,
    mesh=_sc_mesh(),
    out_type=jax.ShapeDtypeStruct((NC, NPAD, H), jnp.float32),
    scratch_types=[
        pltpu.VMEM_SHARED((NPAD, H), jnp.float32),
        pltpu.VMEM((NCH, K), jnp.int32),
        pltpu.VMEM((NCH, K), jnp.int32),
        pltpu.VMEM((K, H), jnp.float32),
        pltpu.VMEM((K, H), jnp.float32),
        pltpu.SemaphoreType.DMA,
        pltpu.SemaphoreType.DMA,
    ],
    compiler_params=pltpu.CompilerParams(use_tc_tiling_on_sc=False),
)
def _agg_kernel(src_hbm, dst_hbm, y_hbm, out_hbm,
                acc_sh, src_v, dst_v, rows0, rows1, semA, semB):
    c = lax.axis_index("c")
    s = lax.axis_index("s")
    tile = c * NS + s

    def fill_row(i, _):
        def fill_lane(k, _):
            rows0[i, pl.ds(k * L, L)] = jnp.zeros((L,), jnp.float32)
            return 0

        lax.fori_loop(0, H // L, fill_lane, 0)
        return 0

    lax.fori_loop(0, K, fill_row, 0)

    def zcopy(m, _):
        pltpu.sync_copy(rows0, acc_sh.at[pl.ds(s * RPT + m * K, K)])
        return 0

    lax.fori_loop(0, RPT // K, zcopy, 0)
    plsc.subcore_barrier()

    pltpu.sync_copy(src_hbm.at[tile], src_v)
    pltpu.sync_copy(dst_hbm.at[tile], dst_v)

    pltpu.async_copy(y_hbm.at[src_v.at[0]], rows0, semA)

    def body(i, _):
        a = 2 * i
        pltpu.make_async_copy(y_hbm.at[src_v.at[a]], rows0, semA).wait()
        pltpu.async_copy(y_hbm.at[src_v.at[a + 1]], rows1, semB)
        pltpu.sync_copy(rows0, acc_sh.at[dst_v.at[a]], add=True)
        pltpu.make_async_copy(y_hbm.at[src_v.at[a + 1]], rows1, semB).wait()

        @pl.when(a + 2 < NCH)
        def _():
            pltpu.async_copy(y_hbm.at[src_v.at[a + 2]], rows0, semA)

        pltpu.sync_copy(rows1, acc_sh.at[dst_v.at[a + 1]], add=True)
        return 0

    lax.fori_loop(0, NCH // 2, body, 0)
    pltpu.make_async_copy(y_hbm.at[src_v.at[NCH - 1]], rows0, semA).wait()
    pltpu.sync_copy(rows0, acc_sh.at[dst_v.at[NCH - 1]], add=True)

    plsc.subcore_barrier()
    pltpu.sync_copy(acc_sh.at[pl.ds(s * RPT, RPT)],
                    out_hbm.at[c, pl.ds(s * RPT, RPT)])


def _mid_body(x_ref, w1_ref, degp_ref, y_ref):
    xw = jnp.dot(x_ref[...], w1_ref[...], preferred_element_type=jnp.float32)
    deg = degp_ref[0, :, 0:1] + degp_ref[1, :, 0:1] + 1.0
    dis = lax.rsqrt(deg)
    y_ref[...] = xw * dis


def _head_body(degp_ref, accp_ref, y_ref, b1_ref, w2_ref, b2_ref, out_ref):
    deg = degp_ref[0, :, 0:1] + degp_ref[1, :, 0:1] + 1.0
    dis = lax.rsqrt(deg)
    acc = accp_ref[0] + accp_ref[1] + y_ref[...]
    h = jnp.maximum(dis * acc + b1_ref[...], 0.0)
    out_ref[...] = jnp.sum(h * w2_ref[...], axis=1, keepdims=True) + b2_ref[...]


def kernel(x, edge_index, W1, b1, W2, b2):
    src_r = edge_index[0].reshape(NW, NCH, K)
    dst_r = edge_index[1].reshape(NW, NCH, K)
    x_pad = jnp.zeros((NPAD, D), jnp.float32).at[:N].set(x)

    deg_parts = _deg_kernel(dst_r)

    y = pl.pallas_call(
        _mid_body,
        grid=(NPAD // BR,),
        in_specs=[
            pl.BlockSpec((BR, D), lambda i: (i, 0)),
            pl.BlockSpec((D, H), lambda i: (0, 0)),
            pl.BlockSpec((NC, BR, L), lambda i: (0, i, 0)),
        ],
        out_specs=pl.BlockSpec((BR, H), lambda i: (i, 0)),
        out_shape=jax.ShapeDtypeStruct((NPAD, H), jnp.float32),
    )(x_pad, W1, deg_parts)

    acc_parts = _agg_kernel(src_r, dst_r, y)

    out_pad = pl.pallas_call(
        _head_body,
        grid=(NPAD // BR,),
        in_specs=[
            pl.BlockSpec((NC, BR, L), lambda i: (0, i, 0)),
            pl.BlockSpec((NC, BR, H), lambda i: (0, i, 0)),
            pl.BlockSpec((BR, H), lambda i: (i, 0)),
            pl.BlockSpec((1, H), lambda i: (0, 0)),
            pl.BlockSpec((1, H), lambda i: (0, 0)),
            pl.BlockSpec((1, 1), lambda i: (0, 0)),
        ],
        out_specs=pl.BlockSpec((BR, 1), lambda i: (i, 0)),
        out_shape=jax.ShapeDtypeStruct((NPAD, 1), jnp.float32),
    )(deg_parts, acc_parts, y, b1.reshape(1, H), W2.reshape(1, H),
      b2.reshape(1, 1))

    return out_pad[:N, 0]
